# baseline (speedup 1.0000x reference)
import jax
import jax.numpy as jnp
from jax.experimental import pallas as pl
from jax.experimental.pallas import tpu as pltpu

_HEAD = 8
_TAIL_SPLITS = 2


def _body(emb_ref, anchor_ref, out_ref, head_sem, tail_sem, anchor_sem):
    B, S, D = out_ref.shape
    tail_rows = S - _HEAD
    base = tail_rows // (8 * _TAIL_SPLITS) * 8
    heads = []
    for b in range(B):
        cp = pltpu.make_async_copy(
            emb_ref.at[b, pl.ds(0, _HEAD), :],
            out_ref.at[b, pl.ds(0, _HEAD), :],
            head_sem.at[b],
        )
        cp.start()
        heads.append(cp)
    tails = []
    for b in range(B):
        for c in range(_TAIL_SPLITS):
            start = _HEAD + c * base
            n = base if c < _TAIL_SPLITS - 1 else tail_rows - (_TAIL_SPLITS - 1) * base
            cp = pltpu.make_async_copy(
                emb_ref.at[b, pl.ds(start, n), :],
                out_ref.at[b, pl.ds(start, n), :],
                tail_sem.at[b * _TAIL_SPLITS + c],
            )
            cp.start()
            tails.append(cp)
    anchors = []
    for b in range(B):
        heads[b].wait()
        cp = pltpu.make_async_copy(
            anchor_ref.at[0, :],
            out_ref.at[b, 0, :],
            anchor_sem.at[b],
        )
        cp.start()
        anchors.append(cp)
    for cp in tails + anchors:
        cp.wait()


@jax.jit
def _run(token_embeddings, style_anchor):
    B, S, D = token_embeddings.shape
    return pl.pallas_call(
        _body,
        in_specs=[
            pl.BlockSpec(memory_space=pltpu.MemorySpace.HBM),
            pl.BlockSpec(memory_space=pltpu.MemorySpace.HBM),
        ],
        out_specs=pl.BlockSpec(memory_space=pltpu.MemorySpace.HBM),
        out_shape=jax.ShapeDtypeStruct((B, S, D), token_embeddings.dtype),
        scratch_shapes=[
            pltpu.SemaphoreType.DMA((B,)),
            pltpu.SemaphoreType.DMA((B * _TAIL_SPLITS,)),
            pltpu.SemaphoreType.DMA((B,)),
        ],
    )(token_embeddings, style_anchor)


def kernel(token_embeddings, style_anchor):
    return _run(token_embeddings, style_anchor)


# speedup vs baseline: 47.2718x; 47.2718x over previous
import functools

import jax
import jax.numpy as jnp
from jax.experimental import pallas as pl
from jax.experimental.pallas import tpu as pltpu


def _body(emb_ref, anchor_ref, out_ref):
    out_ref[...] = emb_ref[...]

    @pl.when(pl.program_id(1) == 0)
    def _():
        out_ref[0, 0, :] = anchor_ref[0, :]


@functools.partial(jax.jit, static_argnames=("rows_per_block",))
def _run(token_embeddings, style_anchor, rows_per_block=1024):
    B, S, D = token_embeddings.shape
    grid = (B, S // rows_per_block)
    return pl.pallas_call(
        _body,
        grid=grid,
        in_specs=[
            pl.BlockSpec((1, rows_per_block, D), lambda b, j: (b, j, 0)),
            pl.BlockSpec((1, D), lambda b, j: (0, 0)),
        ],
        out_specs=pl.BlockSpec((1, rows_per_block, D), lambda b, j: (b, j, 0)),
        out_shape=jax.ShapeDtypeStruct((B, S, D), token_embeddings.dtype),
        compiler_params=pltpu.CompilerParams(
            dimension_semantics=("parallel", "parallel"),
        ),
    )(token_embeddings, style_anchor)


def kernel(token_embeddings, style_anchor):
    return _run(token_embeddings, style_anchor)


# speedup vs baseline: 49.1670x; 1.0401x over previous
import functools

import jax
import jax.numpy as jnp
from jax.experimental import pallas as pl
from jax.experimental.pallas import tpu as pltpu


def _body(emb_ref, anchor_ref, out_ref):
    out_ref[...] = emb_ref[...]

    @pl.when(pl.program_id(1) == 0)
    def _():
        out_ref[0, 0, :] = anchor_ref[0, :]


@functools.partial(jax.jit, static_argnames=("rows_per_block",))
def _run(token_embeddings, style_anchor, rows_per_block=2048):
    B, S, D = token_embeddings.shape
    grid = (B, S // rows_per_block)
    return pl.pallas_call(
        _body,
        grid=grid,
        in_specs=[
            pl.BlockSpec((1, rows_per_block, D), lambda b, j: (b, j, 0)),
            pl.BlockSpec((1, D), lambda b, j: (0, 0)),
        ],
        out_specs=pl.BlockSpec((1, rows_per_block, D), lambda b, j: (b, j, 0)),
        out_shape=jax.ShapeDtypeStruct((B, S, D), token_embeddings.dtype),
        compiler_params=pltpu.CompilerParams(
            dimension_semantics=("parallel", "parallel"),
        ),
    )(token_embeddings, style_anchor)


def kernel(token_embeddings, style_anchor):
    return _run(token_embeddings, style_anchor)
